# baseline (speedup 1.0000x reference)
import functools

import jax
import jax.numpy as jnp
from jax import lax
from jax.experimental import pallas as pl
from jax.experimental.pallas import tpu as pltpu
from jax.experimental.pallas import tpu_sc as plsc

_N_E = 8192
_D = 64
_B = 8
_N = 576
_ROWS = _B * _N

_T = 1024
_CB = _N_E // _T


def _k0_normalize(emb_ref, out_ref, ee_ref):
    e = emb_ref[...]
    nrm = jnp.sqrt(jnp.sum(e * e, axis=1, keepdims=True))
    en = e / jnp.maximum(nrm, 1e-12)
    out_ref[:, :_D] = en
    out_ref[:, _D:] = jnp.zeros((_N_E, 128 - _D), jnp.float32)
    ee_ref[...] = jnp.sum(en * en, axis=1, keepdims=True)


def _mega(z_ref, emb_ref, ee_ref, idx_ref, oh_ref, perp_ref,
          zn_ref, nzz_ref, bv_ref, bi_ref, bit_ref, cnt_ref,
          riota_ref, ciota_ref):
    b = pl.program_id(0)
    k = pl.program_id(1)

    @pl.when((b == 0) & (k == 0))
    def _():
        riota_ref[...] = lax.broadcasted_iota(jnp.int32, (_T, _N), 0)
        ciota_ref[...] = lax.broadcasted_iota(jnp.int32, (_N, _T), 1)

    lidx = bit_ref[...] - k * _T
    oh = (lidx == ciota_ref[...]).astype(jnp.float32)
    oh_ref[...] = oh
    colsum = jnp.sum(oh, axis=0, keepdims=True)

    @pl.when(b == 1)
    def _():
        cnt_ref[pl.ds(k, 1), :] = colsum

    @pl.when(b > 1)
    def _():
        cnt_ref[pl.ds(k, 1), :] = cnt_ref[pl.ds(k, 1), :] + colsum

    @pl.when(k == 0)
    def _():
        zt = z_ref[0]
        znrm = jnp.sqrt(jnp.sum(zt * zt, axis=0, keepdims=True))
        zn = zt / jnp.maximum(znrm, 1e-12)
        nzz_ref[...] = -jnp.sum(zn * zn, axis=0, keepdims=True)
        zn_ref[:_D] = zn + zn
        zn_ref[_D:] = jnp.zeros((128 - _D, _N), jnp.float32)

    m = nzz_ref[...]
    cand = jnp.full((1, _N), k * _T, jnp.int32)

    @pl.when(k == 0)
    def _():
        bv_ref[...] = m
        bi_ref[...] = cand

    @pl.when(k > 0)
    def _():
        upd = m >= bv_ref[...]
        bv_ref[...] = jnp.where(upd, m, bv_ref[...])
        bi_ref[...] = jnp.where(upd, cand, bi_ref[...])

    @pl.when(k == _CB - 1)
    def _():
        idx_ref[...] = bi_ref[...][None]
        bit_ref[...] = bi_ref[...].T

    @pl.when((b == _B) & (k == _CB - 1))
    def _():
        p = cnt_ref[...] / float(_ROWS)
        terms = p * jnp.log(p + 1e-10)
        s = jnp.sum(terms, axis=(0, 1), keepdims=True)[:1, :1]
        perp_ref[...] = jnp.exp(-s)


def _sc_gather(emb_pad, idx_flat):
    info = plsc.get_sparse_core_info()
    nc, ns = info.num_cores, info.num_subcores
    nw = nc * ns
    bpw = _ROWS // nw
    ch = 72
    nch = bpw // ch
    idx2 = idx_flat.reshape(_ROWS // ch, ch)
    mesh = plsc.VectorSubcoreMesh(core_axis_name="c", subcore_axis_name="s")

    @functools.partial(
        pl.kernel,
        mesh=mesh,
        out_type=jax.ShapeDtypeStruct((_ROWS, 128), jnp.float32),
        scratch_types=[
            pltpu.VMEM((nch, ch), jnp.int32),
            pltpu.VMEM((bpw, 128), jnp.float32),
            pltpu.SemaphoreType.DMA,
        ],
    )
    def gather_k(emb_hbm, idx_hbm, out_hbm, idx_v, rows_v, sem):
        wid = lax.axis_index("s") * nc + lax.axis_index("c")
        base = wid * bpw
        pltpu.sync_copy(idx_hbm.at[pl.ds(wid * nch, nch)], idx_v)
        cps = [
            pltpu.async_copy(
                emb_hbm.at[idx_v.at[j]], rows_v.at[pl.ds(j * ch, ch)], sem
            )
            for j in range(nch)
        ]
        for cp in cps:
            cp.wait()
        pltpu.sync_copy(rows_v, out_hbm.at[pl.ds(base, bpw)])

    return gather_k(emb_pad, idx2)


def kernel(z, embedding):
    emb_pad, ee = pl.pallas_call(
        _k0_normalize,
        out_shape=[
            jax.ShapeDtypeStruct((_N_E, 128), jnp.float32),
            jax.ShapeDtypeStruct((_N_E, 1), jnp.float32),
        ],
    )(embedding)

    idx3, onehot, perp = pl.pallas_call(
        _mega,
        grid=(_B + 1, _CB),
        in_specs=[
            pl.BlockSpec((1, _D, _N), lambda b, k: (jnp.minimum(b, _B - 1), 0, 0)),
            pl.BlockSpec((_N_E, 128), lambda b, k: (0, 0)),
            pl.BlockSpec((_N_E, 1), lambda b, k: (0, 0)),
        ],
        out_specs=[
            pl.BlockSpec((1, 1, _N), lambda b, k: (jnp.minimum(b, _B - 1), 0, 0)),
            pl.BlockSpec(
                (_N, _T),
                lambda b, k: (jnp.maximum(b - 1, 0),
                              jnp.where(b == 0, 0, k)),
            ),
            pl.BlockSpec((1, 1), lambda b, k: (0, 0)),
        ],
        out_shape=[
            jax.ShapeDtypeStruct((_B, 1, _N), jnp.int32),
            jax.ShapeDtypeStruct((_ROWS, _N_E), jnp.float32),
            jax.ShapeDtypeStruct((1, 1), jnp.float32),
        ],
        scratch_shapes=[
            pltpu.VMEM((128, _N), jnp.float32),
            pltpu.VMEM((1, _N), jnp.float32),
            pltpu.VMEM((1, _N), jnp.float32),
            pltpu.VMEM((1, _N), jnp.int32),
            pltpu.VMEM((_N, 1), jnp.int32),
            pltpu.VMEM((_CB, _T), jnp.float32),
            pltpu.VMEM((_T, _N), jnp.int32),
            pltpu.VMEM((_N, _T), jnp.int32),
        ],
    )(z, emb_pad, ee)
    indices = idx3.reshape(_ROWS)

    zq = _sc_gather(emb_pad, indices)[:, :_D]
    quant = zq.reshape(_B, _N, _D).transpose(0, 2, 1)

    zero = jnp.float32(0.0)
    return (quant, zero, zero, zero, zero, perp.reshape(()), onehot, indices)


# speedup vs baseline: 1.8902x; 1.8902x over previous
import functools

import jax
import jax.numpy as jnp
from jax import lax
from jax.experimental import pallas as pl
from jax.experimental.pallas import tpu as pltpu
from jax.experimental.pallas import tpu_sc as plsc

_N_E = 8192
_D = 64
_B = 8
_N = 576
_ROWS = _B * _N

_T = 1024
_CB = _N_E // _T


def _mega(z_ref, emb_ref, idx_ref, oh_ref, ep_ref, perp_ref,
          zn_ref, nzz_ref, bv_ref, bi_ref, bit_ref, cnt_ref,
          riota_ref, ciota_ref, en_ref, ee_ref):
    b = pl.program_id(0)
    k = pl.program_id(1)

    @pl.when((b == 0) & (k == 0))
    def _():
        riota_ref[...] = lax.broadcasted_iota(jnp.int32, (_T, _N), 0)
        ciota_ref[...] = lax.broadcasted_iota(jnp.int32, (_N, _T), 1)

    @pl.when(b == 0)
    def _():
        et = emb_ref[pl.ds(k * _T, _T), :]
        enrm = jnp.sqrt(jnp.sum(et * et, axis=1, keepdims=True))
        en = et / jnp.maximum(enrm, 1e-12)
        en_ref[pl.ds(k * _T, _T), :_D] = en
        en_ref[pl.ds(k * _T, _T), _D:] = jnp.zeros((_T, 128 - _D),
                                                   jnp.float32)
        ee_ref[pl.ds(k * _T, _T), :] = jnp.sum(en * en, axis=1,
                                               keepdims=True)
        ep_ref[...] = en_ref[pl.ds(k * _T, _T), :]

    lidx = bit_ref[...] - k * _T
    oh = (lidx == ciota_ref[...]).astype(jnp.float32)
    oh_ref[...] = oh
    colsum = jnp.sum(oh, axis=0, keepdims=True)

    @pl.when(b == 1)
    def _():
        cnt_ref[pl.ds(k, 1), :] = colsum

    @pl.when(b > 1)
    def _():
        cnt_ref[pl.ds(k, 1), :] = cnt_ref[pl.ds(k, 1), :] + colsum

    @pl.when(k == 0)
    def _():
        zt = z_ref[0]
        znrm = jnp.sqrt(jnp.sum(zt * zt, axis=0, keepdims=True))
        zn = zt / jnp.maximum(znrm, 1e-12)
        nzz_ref[...] = -jnp.sum(zn * zn, axis=0, keepdims=True)
        zn_ref[:_D] = zn + zn
        zn_ref[_D:] = jnp.zeros((128 - _D, _N), jnp.float32)

    en = en_ref[pl.ds(k * _T, _T), :]
    eet = ee_ref[pl.ds(k * _T, _T), :]
    dots2 = jnp.dot(en, zn_ref[...], preferred_element_type=jnp.float32)
    d = (nzz_ref[...] - eet) + dots2

    m = jnp.max(d, axis=0, keepdims=True)
    cand = jnp.max(jnp.where(d == m, riota_ref[...], -1),
                   axis=0, keepdims=True) + k * _T

    @pl.when(k == 0)
    def _():
        bv_ref[...] = m
        bi_ref[...] = cand

    @pl.when(k > 0)
    def _():
        upd = m >= bv_ref[...]
        bv_ref[...] = jnp.where(upd, m, bv_ref[...])
        bi_ref[...] = jnp.where(upd, cand, bi_ref[...])

    @pl.when(k == _CB - 1)
    def _():
        idx_ref[...] = bi_ref[...][None]
        bit_ref[...] = bi_ref[...].T

    @pl.when((b == _B) & (k == _CB - 1))
    def _():
        p = cnt_ref[...] / float(_ROWS)
        terms = p * jnp.log(p + 1e-10)
        s = jnp.sum(terms, axis=(0, 1), keepdims=True)[:1, :1]
        perp_ref[...] = jnp.exp(-s)


def _sc_gather(emb_pad, idx_flat):
    info = plsc.get_sparse_core_info()
    nc, ns = info.num_cores, info.num_subcores
    nw = nc * ns
    bpw = _ROWS // nw
    ch = 72
    nch = bpw // ch
    idx2 = idx_flat.reshape(_ROWS // ch, ch)
    mesh = plsc.VectorSubcoreMesh(core_axis_name="c", subcore_axis_name="s")

    @functools.partial(
        pl.kernel,
        mesh=mesh,
        out_type=jax.ShapeDtypeStruct((_ROWS, 128), jnp.float32),
        scratch_types=[
            pltpu.VMEM((nch, ch), jnp.int32),
            pltpu.VMEM((bpw, 128), jnp.float32),
            pltpu.SemaphoreType.DMA,
        ],
    )
    def gather_k(emb_hbm, idx_hbm, out_hbm, idx_v, rows_v, sem):
        wid = lax.axis_index("s") * nc + lax.axis_index("c")
        base = wid * bpw
        pltpu.sync_copy(idx_hbm.at[pl.ds(wid * nch, nch)], idx_v)
        cps = [
            pltpu.async_copy(
                emb_hbm.at[idx_v.at[j]], rows_v.at[pl.ds(j * ch, ch)], sem
            )
            for j in range(nch)
        ]
        for cp in cps:
            cp.wait()
        pltpu.sync_copy(rows_v, out_hbm.at[pl.ds(base, bpw)])

    return gather_k(emb_pad, idx2)


def kernel(z, embedding):
    idx3, onehot, emb_pad, perp = pl.pallas_call(
        _mega,
        grid=(_B + 1, _CB),
        in_specs=[
            pl.BlockSpec((1, _D, _N), lambda b, k: (jnp.minimum(b, _B - 1), 0, 0)),
            pl.BlockSpec((_N_E, _D), lambda b, k: (0, 0)),
        ],
        out_specs=[
            pl.BlockSpec((1, 1, _N), lambda b, k: (jnp.minimum(b, _B - 1), 0, 0)),
            pl.BlockSpec(
                (_N, _T),
                lambda b, k: (jnp.maximum(b - 1, 0),
                              jnp.where(b == 0, 0, k)),
            ),
            pl.BlockSpec(
                (_T, 128),
                lambda b, k: (jnp.where(b == 0, k, _CB - 1), 0),
            ),
            pl.BlockSpec((1, 1), lambda b, k: (0, 0)),
        ],
        out_shape=[
            jax.ShapeDtypeStruct((_B, 1, _N), jnp.int32),
            jax.ShapeDtypeStruct((_ROWS, _N_E), jnp.float32),
            jax.ShapeDtypeStruct((_N_E, 128), jnp.float32),
            jax.ShapeDtypeStruct((1, 1), jnp.float32),
        ],
        scratch_shapes=[
            pltpu.VMEM((128, _N), jnp.float32),
            pltpu.VMEM((1, _N), jnp.float32),
            pltpu.VMEM((1, _N), jnp.float32),
            pltpu.VMEM((1, _N), jnp.int32),
            pltpu.VMEM((_N, 1), jnp.int32),
            pltpu.VMEM((_CB, _T), jnp.float32),
            pltpu.VMEM((_T, _N), jnp.int32),
            pltpu.VMEM((_N, _T), jnp.int32),
            pltpu.VMEM((_N_E, 128), jnp.float32),
            pltpu.VMEM((_N_E, 1), jnp.float32),
        ],
    )(z, embedding)
    indices = idx3.reshape(_ROWS)

    zq = _sc_gather(emb_pad, indices)[:, :_D]
    quant = zq.reshape(_B, _N, _D).transpose(0, 2, 1)

    zero = jnp.float32(0.0)
    return (quant, zero, zero, zero, zero, perp.reshape(()), onehot, indices)
